# baseline (speedup 1.0000x reference)
import functools

import jax
import jax.numpy as jnp
from jax import lax
from jax.experimental import pallas as pl
from jax.experimental.pallas import tpu as pltpu
from jax.experimental.pallas import tpu_sc as plsc

NC = 2
NS = 16
NW = NC * NS
CHUNK = 128
OPS_PER_STEP = 8
STEP_ROWS = CHUNK * OPS_PER_STEP
PACK = 4


def _gather_body(n_steps, table_hbm, idx_hbm, out_hbm, idx_v, rows_v, sem):
    wid = lax.axis_index("s") * NC + lax.axis_index("c")
    pltpu.sync_copy(idx_hbm.at[wid], idx_v)
    rows_per_w = n_steps * STEP_ROWS

    def step(i, carry):
        copies = []
        for k in range(OPS_PER_STEP):
            j = i * OPS_PER_STEP + k
            copies.append(
                pltpu.async_copy(
                    table_hbm.at[idx_v.at[j]],
                    rows_v.at[pl.ds(k * CHUNK, CHUNK)],
                    sem,
                )
            )
        for c in copies:
            c.wait()
        pltpu.sync_copy(
            rows_v,
            out_hbm.at[pl.ds(wid * rows_per_w + i * STEP_ROWS, STEP_ROWS)],
        )
        return carry

    lax.fori_loop(0, n_steps, step, 0)


def _sc_gather(table, idx3d):
    nw, n_chunks, chunk = idx3d.shape
    assert nw == NW and chunk == CHUNK and n_chunks % OPS_PER_STEP == 0
    n_steps = n_chunks // OPS_PER_STEP
    r = NW * n_chunks * CHUNK
    h = table.shape[1]
    mesh = plsc.VectorSubcoreMesh(core_axis_name="c", subcore_axis_name="s")
    return pl.kernel(
        functools.partial(_gather_body, n_steps),
        out_type=jax.ShapeDtypeStruct((r, h), jnp.float32),
        mesh=mesh,
        scratch_types=[
            pltpu.VMEM((n_chunks, CHUNK), jnp.int32),
            pltpu.VMEM((STEP_ROWS, h), jnp.float32),
            pltpu.SemaphoreType.DMA,
        ],
        compiler_params=pltpu.CompilerParams(use_tc_tiling_on_sc=False),
    )(table, idx3d)


def _proj_body(bblk, s, e, g_ref, pmp_ref, whb_ref, wpb_ref, o_ref):
    posp = lax.dot_general(
        pmp_ref[...], wpb_ref[...], (((1,), (0,)), ((), ())),
        preferred_element_type=jnp.float32, precision=lax.Precision.HIGHEST,
    )
    o = lax.dot_general(
        g_ref[...], whb_ref[...], (((1,), (0,)), ((), ())),
        preferred_element_type=jnp.float32, precision=lax.Precision.HIGHEST,
    )
    sp = s // PACK
    tiled = jnp.broadcast_to(posp[None], (bblk, sp, PACK * e))
    o = o.reshape(bblk, sp, PACK * e) + tiled
    parts = [o[:, :, u * e:(u + 1) * e] for u in range(PACK)]
    o4 = jnp.stack(parts, axis=2)
    o_ref[...] = o4.reshape(bblk, s, e)


def _tc_project(g_packed, pm_packed, whblk, wpblk, b, s, e, bblk):
    rp, lanes = g_packed.shape
    sp = pm_packed.shape[0]
    rows_per_b = s // PACK
    grid = (b // bblk,)
    return pl.pallas_call(
        functools.partial(_proj_body, bblk, s, e),
        grid=grid,
        in_specs=[
            pl.BlockSpec((bblk * rows_per_b, lanes), lambda i: (i, 0)),
            pl.BlockSpec((sp, pm_packed.shape[1]), lambda i: (0, 0)),
            pl.BlockSpec(whblk.shape, lambda i: (0, 0)),
            pl.BlockSpec(wpblk.shape, lambda i: (0, 0)),
        ],
        out_specs=pl.BlockSpec((bblk, s, e), lambda i: (i, 0, 0)),
        out_shape=jax.ShapeDtypeStruct((b, s, e), jnp.float32),
    )(g_packed, pm_packed, whblk, wpblk)


def kernel(xss, hmatrix, pmatrix, W):
    b, s = xss.shape
    v, h = hmatrix.shape
    p = pmatrix.shape[1]
    e = W.shape[0]
    r = b * s

    idx3d = xss.reshape(NW, r // (NW * CHUNK), CHUNK)
    gathered = _sc_gather(hmatrix, idx3d)

    eye = jnp.eye(PACK, dtype=W.dtype)
    whblk = jnp.kron(eye, W[:, :h].T)
    wpblk = jnp.kron(eye, W[:, h:].T)
    g_packed = gathered.reshape(r // PACK, PACK * h)
    pm_packed = pmatrix.reshape(s // PACK, PACK * p)

    bblk = 32
    return _tc_project(g_packed, pm_packed, whblk, wpblk, b, s, e, bblk)


# speedup vs baseline: 1.4596x; 1.4596x over previous
import functools

import jax
import jax.numpy as jnp
from jax import lax
from jax.experimental import pallas as pl
from jax.experimental.pallas import tpu as pltpu
from jax.experimental.pallas import tpu_sc as plsc

NC = 2
NS = 16
NW = NC * NS
CHUNK = 128
OPS_PER_STEP = 8
STEP_ROWS = CHUNK * OPS_PER_STEP
PACK = 4


def _gather_body(n_steps, table_hbm, idx_hbm, out_hbm, idx_v, rows_v, sem):
    wid = lax.axis_index("s") * NC + lax.axis_index("c")
    pltpu.sync_copy(idx_hbm.at[wid], idx_v)
    rows_per_w = n_steps * STEP_ROWS

    def step(i, carry):
        copies = []
        for k in range(OPS_PER_STEP):
            j = i * OPS_PER_STEP + k
            copies.append(
                pltpu.async_copy(
                    table_hbm.at[idx_v.at[j]],
                    rows_v.at[pl.ds(k * CHUNK, CHUNK)],
                    sem,
                )
            )
        for c in copies:
            c.wait()
        pltpu.sync_copy(
            rows_v,
            out_hbm.at[pl.ds(wid * rows_per_w + i * STEP_ROWS, STEP_ROWS)],
        )
        return carry

    lax.fori_loop(0, n_steps, step, 0)


def _sc_gather(table, idx3d):
    nw, n_chunks, chunk = idx3d.shape
    assert nw == NW and chunk == CHUNK and n_chunks % OPS_PER_STEP == 0
    n_steps = n_chunks // OPS_PER_STEP
    r = NW * n_chunks * CHUNK
    h = table.shape[1]
    mesh = plsc.VectorSubcoreMesh(core_axis_name="c", subcore_axis_name="s")
    return pl.kernel(
        functools.partial(_gather_body, n_steps),
        out_type=jax.ShapeDtypeStruct((r, h), jnp.float32),
        mesh=mesh,
        scratch_types=[
            pltpu.VMEM((n_chunks, CHUNK), jnp.int32),
            pltpu.VMEM((STEP_ROWS, h), jnp.float32),
            pltpu.SemaphoreType.DMA,
        ],
        compiler_params=pltpu.CompilerParams(use_tc_tiling_on_sc=False),
    )(table, idx3d)


def _proj_body(reps, g_ref, pmp_ref, whb_ref, wpb_ref, o_ref):
    posp = lax.dot_general(
        pmp_ref[...], wpb_ref[...], (((1,), (0,)), ((), ())),
        preferred_element_type=jnp.float32, precision=lax.Precision.HIGHEST,
    )
    sp = posp.shape[0]
    o = lax.dot_general(
        g_ref[...], whb_ref[...], (((1,), (0,)), ((), ())),
        preferred_element_type=jnp.float32, precision=lax.Precision.HIGHEST,
    )
    tiled = jnp.broadcast_to(posp[None], (reps, sp, posp.shape[1]))
    o_ref[...] = o + tiled.reshape(reps * sp, posp.shape[1])


def _tc_project(g_packed, pm_packed, whblk, wpblk, blk_rows):
    rp, lanes = g_packed.shape
    sp = pm_packed.shape[0]
    assert rp % blk_rows == 0 and blk_rows % sp == 0
    reps = blk_rows // sp
    grid = (rp // blk_rows,)
    return pl.pallas_call(
        functools.partial(_proj_body, reps),
        grid=grid,
        in_specs=[
            pl.BlockSpec((blk_rows, lanes), lambda i: (i, 0)),
            pl.BlockSpec((sp, pm_packed.shape[1]), lambda i: (0, 0)),
            pl.BlockSpec(whblk.shape, lambda i: (0, 0)),
            pl.BlockSpec(wpblk.shape, lambda i: (0, 0)),
        ],
        out_specs=pl.BlockSpec((blk_rows, lanes), lambda i: (i, 0)),
        out_shape=jax.ShapeDtypeStruct((rp, lanes), jnp.float32),
    )(g_packed, pm_packed, whblk, wpblk)


def kernel(xss, hmatrix, pmatrix, W):
    b, s = xss.shape
    v, h = hmatrix.shape
    p = pmatrix.shape[1]
    e = W.shape[0]
    r = b * s

    idx3d = xss.reshape(NW, r // (NW * CHUNK), CHUNK)
    table_packed = lax.optimization_barrier(hmatrix.reshape(v // PACK, PACK * h))
    table_lin = table_packed.reshape(v, h)
    gathered = _sc_gather(table_lin, idx3d)

    eye = jnp.eye(PACK, dtype=W.dtype)
    whblk = jnp.kron(eye, W[:, :h].T)
    wpblk = jnp.kron(eye, W[:, h:].T)
    g_packed = gathered.reshape(r // PACK, PACK * h)
    pm_packed = pmatrix.reshape(s // PACK, PACK * p)

    blk_rows = 6400
    out_packed = _tc_project(g_packed, pm_packed, whblk, wpblk, blk_rows)
    return out_packed.reshape(b, s, e)


# speedup vs baseline: 1.6642x; 1.1401x over previous
import functools

import jax
import jax.numpy as jnp
from jax import lax
from jax.experimental import pallas as pl
from jax.experimental.pallas import tpu as pltpu
from jax.experimental.pallas import tpu_sc as plsc

NC = 2
NS = 16
NW = NC * NS
CHUNK = 128
OPS_PER_STEP = 8
STEP_ROWS = CHUNK * OPS_PER_STEP
PACK = 4


def _gather_body(n_steps, table_hbm, idx_hbm, out_hbm, idx_v, rows_v, sem):
    wid = lax.axis_index("s") * NC + lax.axis_index("c")
    pltpu.sync_copy(idx_hbm.at[wid], idx_v)
    rows_per_w = n_steps * STEP_ROWS

    def step(i, carry):
        copies = []
        for k in range(OPS_PER_STEP):
            j = i * OPS_PER_STEP + k
            copies.append(
                pltpu.async_copy(
                    table_hbm.at[idx_v.at[j]],
                    rows_v.at[pl.ds(k * CHUNK, CHUNK)],
                    sem,
                )
            )
        for c in copies:
            c.wait()
        pltpu.sync_copy(
            rows_v,
            out_hbm.at[pl.ds(wid * rows_per_w + i * STEP_ROWS, STEP_ROWS)],
        )
        return carry

    lax.fori_loop(0, n_steps, step, 0)


def _sc_gather(table, idx3d):
    nw, n_chunks, chunk = idx3d.shape
    assert nw == NW and chunk == CHUNK and n_chunks % OPS_PER_STEP == 0
    n_steps = n_chunks // OPS_PER_STEP
    r = NW * n_chunks * CHUNK
    h = table.shape[1]
    mesh = plsc.VectorSubcoreMesh(core_axis_name="c", subcore_axis_name="s")
    return pl.kernel(
        functools.partial(_gather_body, n_steps),
        out_type=jax.ShapeDtypeStruct((r, h), jnp.float32),
        mesh=mesh,
        scratch_types=[
            pltpu.VMEM((n_chunks, CHUNK), jnp.int32),
            pltpu.VMEM((STEP_ROWS, h), jnp.float32),
            pltpu.SemaphoreType.DMA,
        ],
        compiler_params=pltpu.CompilerParams(use_tc_tiling_on_sc=False),
    )(table, idx3d)


def _repack_body(hmt_ref, o_ref):
    x = hmt_ref[...]
    xt = x.T
    h = x.shape[0]
    x4 = xt.reshape(xt.shape[0] // PACK, PACK, h)
    for u in range(PACK):
        o_ref[:, u * h:(u + 1) * h] = x4[:, u, :]


def _tc_repack(hmt, blk):
    h, v = hmt.shape
    grid = (pl.cdiv(v, blk),)
    return pl.pallas_call(
        _repack_body,
        grid=grid,
        in_specs=[pl.BlockSpec((h, blk), lambda i: (0, i))],
        out_specs=pl.BlockSpec((blk // PACK, PACK * h), lambda i: (i, 0)),
        out_shape=jax.ShapeDtypeStruct((v // PACK, PACK * h), jnp.float32),
    )(hmt)


def _proj_body(reps, g_ref, pmp_ref, whb_ref, wpb_ref, o_ref):
    posp = lax.dot_general(
        pmp_ref[...], wpb_ref[...], (((1,), (0,)), ((), ())),
        preferred_element_type=jnp.float32, precision=lax.Precision.HIGHEST,
    )
    sp = posp.shape[0]
    o = lax.dot_general(
        g_ref[...], whb_ref[...], (((1,), (0,)), ((), ())),
        preferred_element_type=jnp.float32, precision=lax.Precision.HIGHEST,
    )
    tiled = jnp.broadcast_to(posp[None], (reps, sp, posp.shape[1]))
    o_ref[...] = o + tiled.reshape(reps * sp, posp.shape[1])


def _tc_project(g_packed, pm_packed, whblk, wpblk, blk_rows):
    rp, lanes = g_packed.shape
    sp = pm_packed.shape[0]
    assert rp % blk_rows == 0 and blk_rows % sp == 0
    reps = blk_rows // sp
    grid = (rp // blk_rows,)
    return pl.pallas_call(
        functools.partial(_proj_body, reps),
        grid=grid,
        in_specs=[
            pl.BlockSpec((blk_rows, lanes), lambda i: (i, 0)),
            pl.BlockSpec((sp, pm_packed.shape[1]), lambda i: (0, 0)),
            pl.BlockSpec(whblk.shape, lambda i: (0, 0)),
            pl.BlockSpec(wpblk.shape, lambda i: (0, 0)),
        ],
        out_specs=pl.BlockSpec((blk_rows, lanes), lambda i: (i, 0)),
        out_shape=jax.ShapeDtypeStruct((rp, lanes), jnp.float32),
    )(g_packed, pm_packed, whblk, wpblk)


def kernel(xss, hmatrix, pmatrix, W):
    b, s = xss.shape
    v, h = hmatrix.shape
    p = pmatrix.shape[1]
    e = W.shape[0]
    r = b * s

    idx3d = xss.reshape(NW, r // (NW * CHUNK), CHUNK)
    table_packed = _tc_repack(hmatrix.T, 16384)
    table_lin = table_packed.reshape(v, h)
    gathered = _sc_gather(table_lin, idx3d)

    eye = jnp.eye(PACK, dtype=W.dtype)
    whblk = jnp.kron(eye, W[:, :h].T)
    wpblk = jnp.kron(eye, W[:, h:].T)
    g_packed = gathered.reshape(r // PACK, PACK * h)
    pm_packed = pmatrix.reshape(s // PACK, PACK * p)

    blk_rows = 6400
    out_packed = _tc_project(g_packed, pm_packed, whblk, wpblk, blk_rows)
    return out_packed.reshape(b, s, e)
